# Initial kernel scaffold; baseline (speedup 1.0000x reference)
import jax
import jax.numpy as jnp
from jax.experimental import pallas as pl


def kernel(indices, values, b, w_weight, v_weight):
    raise NotImplementedError("write your pallas kernel here")


# speedup vs baseline: 1.3271x; 1.3271x over previous
import functools

import jax
import jax.numpy as jnp
from jax import lax
from jax.experimental import pallas as pl
from jax.experimental.pallas import tpu as pltpu
from jax.experimental.pallas import tpu_sc as plsc

B = 16384
F = 26
V = 1000000
K = 16
NW = 32
ROWS_W = B // NW
IDX_COLS = 128
IDXROWS_W = ROWS_W * F // IDX_COLS
CHUNK_BROWS = 128
CHUNK_G = CHUNK_BROWS * F // IDX_COLS
CHUNK_FLAT = CHUNK_BROWS * F
N_CHUNKS = ROWS_W // CHUNK_BROWS


def _fm_body(idx_hbm, vals_hbm, b_hbm, w_hbm, v_hbm, out_hbm,
             idx_v, vals_v, w_v, v_v, out_v, b_v, sem):
    wid = lax.axis_index("c") * 16 + lax.axis_index("s")

    pltpu.sync_copy(idx_hbm.at[pl.ds(wid * IDXROWS_W, IDXROWS_W)], idx_v)
    pltpu.sync_copy(b_hbm, b_v)
    b_vec = b_v[...]

    iota = lax.iota(jnp.int32, 16)
    zeros_i = jnp.zeros((16,), jnp.int32)
    hi_mask = jnp.where(iota >= 6, 1.0, 0.0).astype(jnp.float32)

    def chunk_body(c, carry):
        row0 = wid * ROWS_W + c * CHUNK_BROWS
        pltpu.sync_copy(vals_hbm.at[pl.ds(row0, CHUNK_BROWS)], vals_v)
        handles = []
        for g in range(CHUNK_G):
            isl = idx_v.at[c * CHUNK_G + g]
            handles.append(pltpu.async_copy(
                v_hbm.at[isl], v_v.at[pl.ds(g * IDX_COLS, IDX_COLS)], sem))
            handles.append(pltpu.async_copy(
                w_hbm.at[isl], w_v.at[pl.ds(g * IDX_COLS, IDX_COLS)], sem))
        for h in handles:
            h.wait()

        def rg_body(rg, inner_carry):
            res = jnp.zeros((16,), jnp.float32)
            for rl in range(16):
                r = rg * 16 + rl
                rbase = r * F
                vals0 = vals_v[r, 0:16]
                vals1 = vals_v[r, 10:26]
                acc = jnp.zeros((16,), jnp.float32)
                acc2 = jnp.zeros((16,), jnp.float32)
                for f in range(F):
                    rowv = v_v[rbase + f, :]
                    if f < 16:
                        vb = jnp.take_along_axis(
                            vals0, jnp.full((16,), f, jnp.int32), axis=0,
                            mode="promise_in_bounds")
                    else:
                        vb = jnp.take_along_axis(
                            vals1, jnp.full((16,), f - 10, jnp.int32), axis=0,
                            mode="promise_in_bounds")
                    t = rowv * vb
                    acc = acc + t
                    acc2 = acc2 + t * t
                wv0 = plsc.load_gather(w_v, [rbase + iota])
                wv1 = plsc.load_gather(w_v, [rbase + 10 + iota])
                linv = wv0 * vals0 + wv1 * vals1 * hi_mask
                s = 0.5 * jnp.sum(acc * acc - acc2) + jnp.sum(linv)
                res = jnp.where(iota == rl, jnp.full((16,), s, jnp.float32),
                                res)
            out_v[pl.ds((c * (CHUNK_BROWS // 16) + rg) * 16, 16)] = res + b_vec
            return inner_carry

        lax.fori_loop(0, CHUNK_BROWS // 16, rg_body, 0)
        return carry

    lax.fori_loop(0, N_CHUNKS, chunk_body, 0)
    pltpu.sync_copy(out_v, out_hbm.at[pl.ds(wid * ROWS_W, ROWS_W)])


@jax.jit
def kernel(indices, values, b, w_weight, v_weight):
    idx2d = indices.reshape(B * F // IDX_COLS, IDX_COLS).astype(jnp.int32)
    b16 = jnp.broadcast_to(b.astype(jnp.float32), (16,))
    w1d = w_weight.reshape(V)
    fm = pl.kernel(
        _fm_body,
        out_type=jax.ShapeDtypeStruct((B,), jnp.float32),
        mesh=plsc.VectorSubcoreMesh(core_axis_name="c", subcore_axis_name="s"),
        compiler_params=pltpu.CompilerParams(
            needs_layout_passes=False, use_tc_tiling_on_sc=False),
        scratch_types=[
            pltpu.VMEM((IDXROWS_W, IDX_COLS), jnp.int32),
            pltpu.VMEM((CHUNK_BROWS, F), jnp.float32),
            pltpu.VMEM((CHUNK_FLAT,), jnp.float32),
            pltpu.VMEM((CHUNK_FLAT, K), jnp.float32),
            pltpu.VMEM((ROWS_W,), jnp.float32),
            pltpu.VMEM((16,), jnp.float32),
            pltpu.SemaphoreType.DMA,
        ],
    )
    return fm(idx2d, values, b16, w1d, v_weight)
